# baseline (speedup 1.0000x reference)
import functools

import jax
import jax.numpy as jnp
from jax import lax
from jax.experimental import pallas as pl
from jax.experimental.pallas import tpu as pltpu
from jax.experimental.pallas import tpu_sc as plsc

_NSLOT = 3


def _chunk_schedule(b_per_w):
    if b_per_w == 1024:
        return [64, 208, 336, 336, 80]
    ch = 256
    while b_per_w % ch:
        ch //= 2
    return [ch] * (b_per_w // ch)


def _gather_kernel(B, D, NC, NW, b_per_w):
    chunks = _chunk_schedule(b_per_w)
    offs = [sum(chunks[:i]) for i in range(len(chunks))]
    n_ch = len(chunks)
    ch_max = max(chunks)
    mesh = plsc.VectorSubcoreMesh(core_axis_name="c", subcore_axis_name="s")

    scratch = [pltpu.VMEM((b_per_w,), jnp.int32)]
    scratch += [pltpu.VMEM((ch_max, D), jnp.float32) for _ in range(_NSLOT)]
    scratch += [pltpu.SemaphoreType.DMA for _ in range(2 * _NSLOT)]

    @functools.partial(
        pl.kernel,
        mesh=mesh,
        out_type=[
            jax.ShapeDtypeStruct((B, D), jnp.float32),
            jax.ShapeDtypeStruct((B, D), jnp.float32),
        ],
        scratch_types=scratch,
    )
    def k(ids_hbm, s_hbm, t_hbm, zs_hbm, zt_hbm, idx_v, *bufs):
        rows = bufs[0:_NSLOT]
        sem_g = bufs[_NSLOT:2 * _NSLOT]
        sem_w = bufs[2 * _NSLOT:]

        wid = lax.axis_index("s") * NC + lax.axis_index("c")
        pair = wid // 2
        base = pair * b_per_w
        pltpu.sync_copy(ids_hbm.at[pl.ds(base, chunks[0])],
                        idx_v.at[pl.ds(0, chunks[0])])

        def body(tbl_hbm, out_hbm):
            def start_gather(c):
                slot = c % _NSLOT
                idx_c = idx_v.at[pl.ds(offs[c], chunks[c])]
                dst = rows[slot].at[pl.ds(0, chunks[c])]
                return pltpu.async_copy(tbl_hbm.at[idx_c], dst, sem_g[slot])

            gathers = [None] * n_ch
            writes = [None] * n_ch
            gathers[0] = start_gather(0)
            if n_ch > 1:
                rest = b_per_w - chunks[0]
                pltpu.sync_copy(
                    ids_hbm.at[pl.ds(base + chunks[0], rest)],
                    idx_v.at[pl.ds(chunks[0], rest)])
            for c in range(n_ch):
                slot = c % _NSLOT
                if c + 1 < n_ch:
                    if c + 1 >= _NSLOT:
                        writes[c + 1 - _NSLOT].wait()
                    gathers[c + 1] = start_gather(c + 1)
                gathers[c].wait()
                src = rows[slot].at[pl.ds(0, chunks[c])]
                dst = out_hbm.at[pl.ds(base + offs[c], chunks[c])]
                writes[c] = pltpu.async_copy(src, dst, sem_w[slot])
            for c in range(max(0, n_ch - _NSLOT), n_ch):
                writes[c].wait()

        @pl.when(wid % 2 == 0)
        def _():
            body(s_hbm, zs_hbm)

        @pl.when(wid % 2 == 1)
        def _():
            body(t_hbm, zt_hbm)

    return k


def kernel(object_ids, shape_table, texture_table):
    B = object_ids.shape[0]
    D = shape_table.shape[1]
    info = plsc.get_sparse_core_info()
    NC, NS = info.num_cores, info.num_subcores
    NW = NC * NS
    b_per_w = (2 * B) // NW

    ids = object_ids.astype(jnp.int32)
    k = _gather_kernel(B, D, NC, NW, b_per_w)
    z_s, z_t = k(ids, shape_table, texture_table)
    return (z_s, z_t)


# speedup vs baseline: 1.0063x; 1.0063x over previous
import functools

import jax
import jax.numpy as jnp
from jax import lax
from jax.experimental import pallas as pl
from jax.experimental.pallas import tpu as pltpu
from jax.experimental.pallas import tpu_sc as plsc

_NSLOT = 3


def _chunk_schedule(b_per_w):
    if b_per_w == 1024:
        return [64, 256, 256, 256, 192]
    ch = 256
    while b_per_w % ch:
        ch //= 2
    return [ch] * (b_per_w // ch)


def _gather_kernel(B, D, NC, NW, b_per_w):
    chunks = _chunk_schedule(b_per_w)
    offs = [sum(chunks[:i]) for i in range(len(chunks))]
    n_ch = len(chunks)
    ch_max = max(chunks)
    mesh = plsc.VectorSubcoreMesh(core_axis_name="c", subcore_axis_name="s")

    scratch = [pltpu.VMEM((b_per_w,), jnp.int32)]
    scratch += [pltpu.VMEM((ch_max, D), jnp.float32) for _ in range(_NSLOT)]
    scratch += [pltpu.SemaphoreType.DMA for _ in range(2 * _NSLOT)]

    @functools.partial(
        pl.kernel,
        mesh=mesh,
        out_type=[
            jax.ShapeDtypeStruct((B, D), jnp.float32),
            jax.ShapeDtypeStruct((B, D), jnp.float32),
        ],
        scratch_types=scratch,
    )
    def k(ids_hbm, s_hbm, t_hbm, zs_hbm, zt_hbm, idx_v, *bufs):
        rows = bufs[0:_NSLOT]
        sem_g = bufs[_NSLOT:2 * _NSLOT]
        sem_w = bufs[2 * _NSLOT:]

        wid = lax.axis_index("s") * NC + lax.axis_index("c")
        pair = wid // 2
        base = pair * b_per_w
        pltpu.sync_copy(ids_hbm.at[pl.ds(base, chunks[0])],
                        idx_v.at[pl.ds(0, chunks[0])])

        def body(tbl_hbm, out_hbm):
            def start_gather(c):
                slot = c % _NSLOT
                idx_c = idx_v.at[pl.ds(offs[c], chunks[c])]
                dst = rows[slot].at[pl.ds(0, chunks[c])]
                return pltpu.async_copy(tbl_hbm.at[idx_c], dst, sem_g[slot])

            gathers = [None] * n_ch
            writes = [None] * n_ch
            gathers[0] = start_gather(0)
            if n_ch > 1:
                rest = b_per_w - chunks[0]
                pltpu.sync_copy(
                    ids_hbm.at[pl.ds(base + chunks[0], rest)],
                    idx_v.at[pl.ds(chunks[0], rest)])
            for c in range(n_ch):
                slot = c % _NSLOT
                if c + 1 < n_ch:
                    if c + 1 >= _NSLOT:
                        writes[c + 1 - _NSLOT].wait()
                    gathers[c + 1] = start_gather(c + 1)
                gathers[c].wait()
                src = rows[slot].at[pl.ds(0, chunks[c])]
                dst = out_hbm.at[pl.ds(base + offs[c], chunks[c])]
                writes[c] = pltpu.async_copy(src, dst, sem_w[slot])
            for c in range(max(0, n_ch - _NSLOT), n_ch):
                writes[c].wait()

        @pl.when(wid % 2 == 0)
        def _():
            body(s_hbm, zs_hbm)

        @pl.when(wid % 2 == 1)
        def _():
            body(t_hbm, zt_hbm)

    return k


def kernel(object_ids, shape_table, texture_table):
    B = object_ids.shape[0]
    D = shape_table.shape[1]
    info = plsc.get_sparse_core_info()
    NC, NS = info.num_cores, info.num_subcores
    NW = NC * NS
    b_per_w = (2 * B) // NW

    ids = object_ids.astype(jnp.int32)
    k = _gather_kernel(B, D, NC, NW, b_per_w)
    z_s, z_t = k(ids, shape_table, texture_table)
    return (z_s, z_t)


# speedup vs baseline: 1.0199x; 1.0135x over previous
import functools

import jax
import jax.numpy as jnp
from jax import lax
from jax.experimental import pallas as pl
from jax.experimental.pallas import tpu as pltpu
from jax.experimental.pallas import tpu_sc as plsc

_NSLOT = 3


def _gather_kernel(B, D, NC, NW, b_per_w, CH):
    n_ch = b_per_w // CH
    mesh = plsc.VectorSubcoreMesh(core_axis_name="c", subcore_axis_name="s")

    scratch = [pltpu.VMEM((b_per_w,), jnp.int32)]
    scratch += [pltpu.VMEM((CH, D), jnp.float32) for _ in range(_NSLOT)]
    scratch += [pltpu.SemaphoreType.DMA for _ in range(2 * _NSLOT)]

    @functools.partial(
        pl.kernel,
        mesh=mesh,
        out_type=[
            jax.ShapeDtypeStruct((B, D), jnp.float32),
            jax.ShapeDtypeStruct((B, D), jnp.float32),
        ],
        scratch_types=scratch,
    )
    def k(ids_hbm, s_hbm, t_hbm, zs_hbm, zt_hbm, idx_v, *bufs):
        rows = bufs[0:_NSLOT]
        sem_g = bufs[_NSLOT:2 * _NSLOT]
        sem_w = bufs[2 * _NSLOT:]

        wid = lax.axis_index("s") * NC + lax.axis_index("c")
        pair = wid // 2
        base = pair * b_per_w
        pltpu.sync_copy(ids_hbm.at[pl.ds(base, CH)], idx_v.at[pl.ds(0, CH)])

        def body(tbl_hbm, out_hbm):
            def start_gather(c):
                slot = c % _NSLOT
                idx_c = idx_v.at[pl.ds(c * CH, CH)]
                return pltpu.async_copy(tbl_hbm.at[idx_c], rows[slot], sem_g[slot])

            gathers = [None] * n_ch
            writes = [None] * n_ch
            gathers[0] = start_gather(0)
            if n_ch > 1:
                pltpu.sync_copy(
                    ids_hbm.at[pl.ds(base + CH, b_per_w - CH)],
                    idx_v.at[pl.ds(CH, b_per_w - CH)])
            for c in range(n_ch):
                slot = c % _NSLOT
                if c + 1 < n_ch:
                    if c + 1 >= _NSLOT:
                        writes[c + 1 - _NSLOT].wait()
                    gathers[c + 1] = start_gather(c + 1)
                gathers[c].wait()
                dst = pl.ds(base + c * CH, CH)
                writes[c] = pltpu.async_copy(rows[slot], out_hbm.at[dst], sem_w[slot])
            for c in range(max(0, n_ch - _NSLOT), n_ch):
                writes[c].wait()

        @pl.when(wid % 2 == 0)
        def _():
            body(s_hbm, zs_hbm)

        @pl.when(wid % 2 == 1)
        def _():
            body(t_hbm, zt_hbm)

    return k


def kernel(object_ids, shape_table, texture_table):
    B = object_ids.shape[0]
    D = shape_table.shape[1]
    info = plsc.get_sparse_core_info()
    NC, NS = info.num_cores, info.num_subcores
    NW = NC * NS
    b_per_w = (2 * B) // NW
    CH = 256

    ids = object_ids.astype(jnp.int32)
    k = _gather_kernel(B, D, NC, NW, b_per_w, CH)
    z_s, z_t = k(ids, shape_table, texture_table)
    return (z_s, z_t)
